# Initial kernel scaffold; baseline (speedup 1.0000x reference)
import jax
import jax.numpy as jnp
from jax.experimental import pallas as pl


def kernel(x, edge_index, W1, b1, W2, b2):
    raise NotImplementedError("write your pallas kernel here")


# speedup vs baseline: 16.9579x; 16.9579x over previous
import functools

import jax
import jax.numpy as jnp
from jax import lax
from jax.experimental import pallas as pl
from jax.experimental.pallas import tpu as pltpu
from jax.experimental.pallas import tpu_sc as plsc

N_NODES = 10000
N_EDGES = 320000
D_IN = 128
D_HID = 64
D_OUT = 32

NC = 2
NS = 16
NW = NC * NS
BLK = 128
NB = 10240 // BLK
E_PAD = NW * NB * BLK
N_PAD = 10240
ROWS_PER_SUB = N_PAD // NS
MM_BLK = 512
GRID = N_PAD // MM_BLK

_mesh = plsc.VectorSubcoreMesh(core_axis_name="core", subcore_axis_name="subcore")
_sc_params = pltpu.CompilerParams(use_tc_tiling_on_sc=False)


def _worker_id():
    return lax.axis_index("core") * NS + lax.axis_index("subcore")


def _zero_fill(ref, width):
    z = jnp.zeros((16,), jnp.float32)

    @pl.loop(0, BLK)
    def _(r):
        for t in range(width // 16):
            ref[r, pl.ds(t * 16, 16)] = z


DEG_W = 16


@functools.partial(
    pl.kernel,
    out_type=jax.ShapeDtypeStruct((NC, N_PAD, DEG_W), jnp.float32),
    mesh=_mesh,
    scratch_types=[
        pltpu.VMEM((NB, BLK), jnp.int32),
        pltpu.VMEM((BLK, DEG_W), jnp.float32),
        pltpu.VMEM((BLK, DEG_W), jnp.float32),
        pltpu.VMEM_SHARED((N_PAD, DEG_W), jnp.float32),
    ],
    compiler_params=_sc_params,
)
def _deg_kernel(dst_hbm, out_hbm, idx_v, ones_v, zero_v, acc_sh):
    c = lax.axis_index("core")
    s = lax.axis_index("subcore")
    wid = _worker_id()

    one = jnp.ones((16,), jnp.float32)

    @pl.loop(0, BLK)
    def _(r):
        ones_v[r, pl.ds(0, 16)] = one

    _zero_fill(zero_v, DEG_W)
    for k in range(ROWS_PER_SUB // BLK):
        pltpu.sync_copy(zero_v, acc_sh.at[pl.ds(s * ROWS_PER_SUB + k * BLK, BLK)])
    pltpu.sync_copy(dst_hbm.at[wid], idx_v)
    plsc.subcore_barrier()

    @pl.loop(0, NB)
    def _(j):
        pltpu.sync_copy(ones_v, acc_sh.at[idx_v.at[j]], add=True)

    plsc.subcore_barrier()
    pltpu.sync_copy(
        acc_sh.at[pl.ds(s * ROWS_PER_SUB, ROWS_PER_SUB)],
        out_hbm.at[c, pl.ds(s * ROWS_PER_SUB, ROWS_PER_SUB)],
    )


def _make_prop(d_feat):
    @functools.partial(
        pl.kernel,
        out_type=jax.ShapeDtypeStruct((NC, N_PAD, d_feat), jnp.float32),
        mesh=_mesh,
        scratch_types=[
            pltpu.VMEM((NB, BLK), jnp.int32),
            pltpu.VMEM((NB, BLK), jnp.int32),
            pltpu.VMEM((BLK, d_feat), jnp.float32),
            pltpu.VMEM((BLK, d_feat), jnp.float32),
            pltpu.VMEM_SHARED((N_PAD, d_feat), jnp.float32),
        ],
        compiler_params=_sc_params,
    )
    def _prop(g_hbm, src_hbm, dst_hbm, out_hbm, src_v, dst_v, rows_v, zero_v, acc_sh):
        c = lax.axis_index("core")
        s = lax.axis_index("subcore")
        wid = _worker_id()

        _zero_fill(zero_v, d_feat)
        for k in range(ROWS_PER_SUB // BLK):
            pltpu.sync_copy(zero_v, acc_sh.at[pl.ds(s * ROWS_PER_SUB + k * BLK, BLK)])
        pltpu.sync_copy(src_hbm.at[wid], src_v)
        pltpu.sync_copy(dst_hbm.at[wid], dst_v)
        plsc.subcore_barrier()

        @pl.loop(0, NB)
        def _(j):
            pltpu.sync_copy(g_hbm.at[src_v.at[j]], rows_v)
            pltpu.sync_copy(rows_v, acc_sh.at[dst_v.at[j]], add=True)

        plsc.subcore_barrier()
        pltpu.sync_copy(
            acc_sh.at[pl.ds(s * ROWS_PER_SUB, ROWS_PER_SUB)],
            out_hbm.at[c, pl.ds(s * ROWS_PER_SUB, ROWS_PER_SUB)],
        )

    return _prop


_prop_hid = _make_prop(D_HID)
_prop_out = _make_prop(D_OUT)


def _mm_body(x_ref, w_ref, o_ref):
    o_ref[...] = jnp.dot(
        x_ref[...], w_ref[...], preferred_element_type=jnp.float32,
        precision=lax.Precision.HIGHEST,
    )


def _matmul(x_p, w):
    d_out = w.shape[1]
    return pl.pallas_call(
        _mm_body,
        grid=(GRID,),
        in_specs=[
            pl.BlockSpec((MM_BLK, x_p.shape[1]), lambda i: (i, 0)),
            pl.BlockSpec(w.shape, lambda i: (0, 0)),
        ],
        out_specs=pl.BlockSpec((MM_BLK, d_out), lambda i: (i, 0)),
        out_shape=jax.ShapeDtypeStruct((N_PAD, d_out), jnp.float32),
    )(x_p, w)


def _prep_body(deg_ref, h_ref, dinv_ref, g_ref):
    deg = deg_ref[0, :, 0:1] + deg_ref[1, :, 0:1] + 1.0
    dinv = lax.rsqrt(deg)
    dinv_ref[...] = dinv
    g_ref[...] = h_ref[...] * dinv


def _prep(deg_part, h1):
    return pl.pallas_call(
        _prep_body,
        grid=(GRID,),
        in_specs=[
            pl.BlockSpec((NC, MM_BLK, DEG_W), lambda i: (0, i, 0)),
            pl.BlockSpec((MM_BLK, D_HID), lambda i: (i, 0)),
        ],
        out_specs=[
            pl.BlockSpec((MM_BLK, 1), lambda i: (i, 0)),
            pl.BlockSpec((MM_BLK, D_HID), lambda i: (i, 0)),
        ],
        out_shape=[
            jax.ShapeDtypeStruct((N_PAD, 1), jnp.float32),
            jax.ShapeDtypeStruct((N_PAD, D_HID), jnp.float32),
        ],
    )(deg_part, h1)


def _mid_body(p_ref, g_ref, dinv_ref, b_ref, w_ref, o_ref):
    pre = (p_ref[0] + p_ref[1] + g_ref[...]) * dinv_ref[...] + b_ref[...]
    h = jnp.maximum(pre, 0.0)
    o_ref[...] = jnp.dot(
        h, w_ref[...], preferred_element_type=jnp.float32,
        precision=lax.Precision.HIGHEST,
    ) * dinv_ref[...]


def _mid(part1, g1, dinv, b1, w2):
    return pl.pallas_call(
        _mid_body,
        grid=(GRID,),
        in_specs=[
            pl.BlockSpec((NC, MM_BLK, D_HID), lambda i: (0, i, 0)),
            pl.BlockSpec((MM_BLK, D_HID), lambda i: (i, 0)),
            pl.BlockSpec((MM_BLK, 1), lambda i: (i, 0)),
            pl.BlockSpec((1, D_HID), lambda i: (0, 0)),
            pl.BlockSpec((D_HID, D_OUT), lambda i: (0, 0)),
        ],
        out_specs=pl.BlockSpec((MM_BLK, D_OUT), lambda i: (i, 0)),
        out_shape=jax.ShapeDtypeStruct((N_PAD, D_OUT), jnp.float32),
    )(part1, g1, dinv, b1, w2)


def _final_body(p_ref, g_ref, dinv_ref, b_ref, o_ref):
    o_ref[...] = (p_ref[0] + p_ref[1] + g_ref[...]) * dinv_ref[...] + b_ref[...]


def _final(part2, g2, dinv, b2):
    return pl.pallas_call(
        _final_body,
        grid=(GRID,),
        in_specs=[
            pl.BlockSpec((NC, MM_BLK, D_OUT), lambda i: (0, i, 0)),
            pl.BlockSpec((MM_BLK, D_OUT), lambda i: (i, 0)),
            pl.BlockSpec((MM_BLK, 1), lambda i: (i, 0)),
            pl.BlockSpec((1, D_OUT), lambda i: (0, 0)),
        ],
        out_specs=pl.BlockSpec((MM_BLK, D_OUT), lambda i: (i, 0)),
        out_shape=jax.ShapeDtypeStruct((N_NODES, D_OUT), jnp.float32),
    )(part2, g2, dinv, b2)


def kernel(x, edge_index, W1, b1, W2, b2):
    ei = edge_index.astype(jnp.int32)
    pad = E_PAD - N_EDGES
    src = jnp.concatenate([ei[0], jnp.zeros((pad,), jnp.int32)]).reshape(NW, NB, BLK)
    dst = jnp.concatenate([ei[1], jnp.full((pad,), N_NODES, jnp.int32)]).reshape(NW, NB, BLK)
    x_p = jnp.concatenate([x, jnp.zeros((N_PAD - N_NODES, D_IN), x.dtype)], axis=0)

    deg_part = _deg_kernel(dst)
    h1 = _matmul(x_p, W1)
    dinv, g1 = _prep(deg_part, h1)
    part1 = _prop_hid(g1, src, dst)
    g2 = _mid(part1, g1, dinv, b1.reshape(1, D_HID), W2)
    part2 = _prop_out(g2, src, dst)
    return _final(part2, g2, dinv, b2.reshape(1, D_OUT))


# speedup vs baseline: 19.4882x; 1.1492x over previous
import functools

import jax
import jax.numpy as jnp
from jax import lax
from jax.experimental import pallas as pl
from jax.experimental.pallas import tpu as pltpu
from jax.experimental.pallas import tpu_sc as plsc

N_NODES = 10000
N_EDGES = 320000
D_IN = 128
D_HID = 64
D_OUT = 32

NC = 2
NS = 16
NW = NC * NS
BLK = 128
NB = 10240 // BLK
E_PAD = NW * NB * BLK
N_PAD = 10240
ROWS_PER_SUB = N_PAD // NS
MM_BLK = 512
GRID = N_PAD // MM_BLK

_mesh = plsc.VectorSubcoreMesh(core_axis_name="core", subcore_axis_name="subcore")
_sc_params = pltpu.CompilerParams(use_tc_tiling_on_sc=False)


def _worker_id():
    return lax.axis_index("core") * NS + lax.axis_index("subcore")


def _zero_fill(ref, width):
    z = jnp.zeros((16,), jnp.float32)

    @pl.loop(0, BLK)
    def _(r):
        for t in range(width // 16):
            ref[r, pl.ds(t * 16, 16)] = z


DEG_W = 16


@functools.partial(
    pl.kernel,
    out_type=jax.ShapeDtypeStruct((NC, N_PAD, DEG_W), jnp.float32),
    mesh=_mesh,
    scratch_types=[
        pltpu.VMEM((NB, BLK), jnp.int32),
        pltpu.VMEM((BLK, DEG_W), jnp.float32),
        pltpu.VMEM((BLK, DEG_W), jnp.float32),
        pltpu.VMEM_SHARED((N_PAD, DEG_W), jnp.float32),
    ],
    compiler_params=_sc_params,
)
def _deg_kernel(dst_hbm, out_hbm, idx_v, ones_v, zero_v, acc_sh):
    c = lax.axis_index("core")
    s = lax.axis_index("subcore")
    wid = _worker_id()

    one = jnp.ones((16,), jnp.float32)

    @pl.loop(0, BLK)
    def _(r):
        ones_v[r, pl.ds(0, 16)] = one

    _zero_fill(zero_v, DEG_W)
    for k in range(ROWS_PER_SUB // BLK):
        pltpu.sync_copy(zero_v, acc_sh.at[pl.ds(s * ROWS_PER_SUB + k * BLK, BLK)])
    pltpu.sync_copy(dst_hbm.at[wid], idx_v)
    plsc.subcore_barrier()

    @pl.loop(0, NB)
    def _(j):
        pltpu.sync_copy(ones_v, acc_sh.at[idx_v.at[j]], add=True)

    plsc.subcore_barrier()
    pltpu.sync_copy(
        acc_sh.at[pl.ds(s * ROWS_PER_SUB, ROWS_PER_SUB)],
        out_hbm.at[c, pl.ds(s * ROWS_PER_SUB, ROWS_PER_SUB)],
    )


NBUF = 4


def _make_prop(d_feat):
    @functools.partial(
        pl.kernel,
        out_type=jax.ShapeDtypeStruct((NC, N_PAD, d_feat), jnp.float32),
        mesh=_mesh,
        scratch_types=[
            pltpu.VMEM((NB, BLK), jnp.int32),
            pltpu.VMEM((NB, BLK), jnp.int32),
            [pltpu.VMEM((BLK, d_feat), jnp.float32)] * NBUF,
            pltpu.VMEM((BLK, d_feat), jnp.float32),
            pltpu.VMEM_SHARED((N_PAD, d_feat), jnp.float32),
            [pltpu.SemaphoreType.DMA] * NBUF,
            [pltpu.SemaphoreType.DMA] * NBUF,
        ],
        compiler_params=_sc_params,
    )
    def _prop(g_hbm, src_hbm, dst_hbm, out_hbm, src_v, dst_v, bufs, zero_v, acc_sh,
              gsems, ssems):
        c = lax.axis_index("core")
        s = lax.axis_index("subcore")
        wid = _worker_id()

        _zero_fill(zero_v, d_feat)
        for k in range(ROWS_PER_SUB // BLK):
            pltpu.sync_copy(zero_v, acc_sh.at[pl.ds(s * ROWS_PER_SUB + k * BLK, BLK)])
        pltpu.sync_copy(src_hbm.at[wid], src_v)
        pltpu.sync_copy(dst_hbm.at[wid], dst_v)
        plsc.subcore_barrier()

        for b in range(NBUF):
            pltpu.async_copy(g_hbm.at[src_v.at[b]], bufs[b], gsems[b])

        @pl.loop(0, NB, step=NBUF)
        def _(j):
            for b in range(NBUF):
                jj = j + b
                pltpu.make_async_copy(g_hbm.at[src_v.at[jj]], bufs[b], gsems[b]).wait()
                pltpu.async_copy(bufs[b], acc_sh.at[dst_v.at[jj]], ssems[b], add=True)
            for b in range(NBUF):
                jn = j + b + NBUF

                @pl.when(jn < NB)
                def _():
                    pltpu.make_async_copy(
                        bufs[b], acc_sh.at[dst_v.at[0]], ssems[b]
                    ).wait()
                    pltpu.async_copy(g_hbm.at[src_v.at[jn]], bufs[b], gsems[b])

        for b in range(NBUF):
            pltpu.make_async_copy(bufs[b], acc_sh.at[dst_v.at[0]], ssems[b]).wait()

        plsc.subcore_barrier()
        pltpu.sync_copy(
            acc_sh.at[pl.ds(s * ROWS_PER_SUB, ROWS_PER_SUB)],
            out_hbm.at[c, pl.ds(s * ROWS_PER_SUB, ROWS_PER_SUB)],
        )

    return _prop


_prop_hid = _make_prop(D_HID)
_prop_out = _make_prop(D_OUT)


def _mm_body(x_ref, w_ref, o_ref):
    o_ref[...] = jnp.dot(
        x_ref[...], w_ref[...], preferred_element_type=jnp.float32,
        precision=lax.Precision.HIGHEST,
    )


def _matmul(x_p, w):
    d_out = w.shape[1]
    return pl.pallas_call(
        _mm_body,
        grid=(GRID,),
        in_specs=[
            pl.BlockSpec((MM_BLK, x_p.shape[1]), lambda i: (i, 0)),
            pl.BlockSpec(w.shape, lambda i: (0, 0)),
        ],
        out_specs=pl.BlockSpec((MM_BLK, d_out), lambda i: (i, 0)),
        out_shape=jax.ShapeDtypeStruct((N_PAD, d_out), jnp.float32),
    )(x_p, w)


def _prep_body(deg_ref, h_ref, dinv_ref, g_ref):
    deg = deg_ref[0, :, 0:1] + deg_ref[1, :, 0:1] + 1.0
    dinv = lax.rsqrt(deg)
    dinv_ref[...] = dinv
    g_ref[...] = h_ref[...] * dinv


def _prep(deg_part, h1):
    return pl.pallas_call(
        _prep_body,
        grid=(GRID,),
        in_specs=[
            pl.BlockSpec((NC, MM_BLK, DEG_W), lambda i: (0, i, 0)),
            pl.BlockSpec((MM_BLK, D_HID), lambda i: (i, 0)),
        ],
        out_specs=[
            pl.BlockSpec((MM_BLK, 1), lambda i: (i, 0)),
            pl.BlockSpec((MM_BLK, D_HID), lambda i: (i, 0)),
        ],
        out_shape=[
            jax.ShapeDtypeStruct((N_PAD, 1), jnp.float32),
            jax.ShapeDtypeStruct((N_PAD, D_HID), jnp.float32),
        ],
    )(deg_part, h1)


def _mid_body(p_ref, g_ref, dinv_ref, b_ref, w_ref, o_ref):
    pre = (p_ref[0] + p_ref[1] + g_ref[...]) * dinv_ref[...] + b_ref[...]
    h = jnp.maximum(pre, 0.0)
    o_ref[...] = jnp.dot(
        h, w_ref[...], preferred_element_type=jnp.float32,
        precision=lax.Precision.HIGHEST,
    ) * dinv_ref[...]


def _mid(part1, g1, dinv, b1, w2):
    return pl.pallas_call(
        _mid_body,
        grid=(GRID,),
        in_specs=[
            pl.BlockSpec((NC, MM_BLK, D_HID), lambda i: (0, i, 0)),
            pl.BlockSpec((MM_BLK, D_HID), lambda i: (i, 0)),
            pl.BlockSpec((MM_BLK, 1), lambda i: (i, 0)),
            pl.BlockSpec((1, D_HID), lambda i: (0, 0)),
            pl.BlockSpec((D_HID, D_OUT), lambda i: (0, 0)),
        ],
        out_specs=pl.BlockSpec((MM_BLK, D_OUT), lambda i: (i, 0)),
        out_shape=jax.ShapeDtypeStruct((N_PAD, D_OUT), jnp.float32),
    )(part1, g1, dinv, b1, w2)


def _final_body(p_ref, g_ref, dinv_ref, b_ref, o_ref):
    o_ref[...] = (p_ref[0] + p_ref[1] + g_ref[...]) * dinv_ref[...] + b_ref[...]


def _final(part2, g2, dinv, b2):
    return pl.pallas_call(
        _final_body,
        grid=(GRID,),
        in_specs=[
            pl.BlockSpec((NC, MM_BLK, D_OUT), lambda i: (0, i, 0)),
            pl.BlockSpec((MM_BLK, D_OUT), lambda i: (i, 0)),
            pl.BlockSpec((MM_BLK, 1), lambda i: (i, 0)),
            pl.BlockSpec((1, D_OUT), lambda i: (0, 0)),
        ],
        out_specs=pl.BlockSpec((MM_BLK, D_OUT), lambda i: (i, 0)),
        out_shape=jax.ShapeDtypeStruct((N_NODES, D_OUT), jnp.float32),
    )(part2, g2, dinv, b2)


def kernel(x, edge_index, W1, b1, W2, b2):
    ei = edge_index.astype(jnp.int32)
    pad = E_PAD - N_EDGES
    src = jnp.concatenate([ei[0], jnp.zeros((pad,), jnp.int32)]).reshape(NW, NB, BLK)
    dst = jnp.concatenate([ei[1], jnp.full((pad,), N_NODES, jnp.int32)]).reshape(NW, NB, BLK)
    x_p = jnp.concatenate([x, jnp.zeros((N_PAD - N_NODES, D_IN), x.dtype)], axis=0)

    deg_part = _deg_kernel(dst)
    h1 = _matmul(x_p, W1)
    dinv, g1 = _prep(deg_part, h1)
    part1 = _prop_hid(g1, src, dst)
    g2 = _mid(part1, g1, dinv, b1.reshape(1, D_HID), W2)
    part2 = _prop_out(g2, src, dst)
    return _final(part2, g2, dinv, b2.reshape(1, D_OUT))


# speedup vs baseline: 31.1453x; 1.5982x over previous
import functools

import jax
import jax.numpy as jnp
from jax import lax
from jax.experimental import pallas as pl
from jax.experimental.pallas import tpu as pltpu
from jax.experimental.pallas import tpu_sc as plsc

N_NODES = 10000
N_EDGES = 320000
D_IN = 128
D_HID = 64
D_OUT = 32

NC = 2
NS = 16
NW = NC * NS
BLK = 128
NB = 10240 // BLK
E_PAD = NW * NB * BLK
N_PAD = 10016
ROWS_PER_SUB = N_PAD // NS
MM_BLK = 512
GRID = (N_PAD + MM_BLK - 1) // MM_BLK

_ZCHUNKS = [(o, min(BLK, ROWS_PER_SUB - o)) for o in range(0, ROWS_PER_SUB, BLK)]

_mesh = plsc.VectorSubcoreMesh(core_axis_name="core", subcore_axis_name="subcore")
_sc_params = pltpu.CompilerParams(use_tc_tiling_on_sc=False)


def _worker_id():
    return lax.axis_index("core") * NS + lax.axis_index("subcore")


def _zero_fill(ref, width):
    z = jnp.zeros((16,), jnp.float32)

    @pl.loop(0, BLK)
    def _(r):
        for t in range(width // 16):
            ref[r, pl.ds(t * 16, 16)] = z


DEG_W = 16


@functools.partial(
    pl.kernel,
    out_type=jax.ShapeDtypeStruct((NC, N_PAD, DEG_W), jnp.float32),
    mesh=_mesh,
    scratch_types=[
        pltpu.VMEM((NB, BLK), jnp.int32),
        pltpu.VMEM((BLK, DEG_W), jnp.float32),
        pltpu.VMEM((BLK, DEG_W), jnp.float32),
        pltpu.VMEM_SHARED((N_PAD, DEG_W), jnp.float32),
    ],
    compiler_params=_sc_params,
)
def _deg_kernel(dst_hbm, out_hbm, idx_v, ones_v, zero_v, acc_sh):
    c = lax.axis_index("core")
    s = lax.axis_index("subcore")
    wid = _worker_id()

    one = jnp.ones((16,), jnp.float32)

    @pl.loop(0, BLK)
    def _(r):
        ones_v[r, pl.ds(0, 16)] = one

    _zero_fill(zero_v, DEG_W)
    for o, sz in _ZCHUNKS:
        pltpu.sync_copy(
            zero_v.at[pl.ds(0, sz)], acc_sh.at[pl.ds(s * ROWS_PER_SUB + o, sz)]
        )
    pltpu.sync_copy(dst_hbm.at[wid], idx_v)
    plsc.subcore_barrier()

    @pl.loop(0, NB)
    def _(j):
        pltpu.sync_copy(ones_v, acc_sh.at[idx_v.at[j]], add=True)

    plsc.subcore_barrier()
    pltpu.sync_copy(
        acc_sh.at[pl.ds(s * ROWS_PER_SUB, ROWS_PER_SUB)],
        out_hbm.at[c, pl.ds(s * ROWS_PER_SUB, ROWS_PER_SUB)],
    )


NBUF = 4


def _make_prop(d_feat, table_in_spmem):
    scratch = [
        pltpu.VMEM((NB, BLK), jnp.int32),
        pltpu.VMEM((NB, BLK), jnp.int32),
        [pltpu.VMEM((BLK, d_feat), jnp.float32)] * NBUF,
        pltpu.VMEM((BLK, d_feat), jnp.float32),
        pltpu.VMEM_SHARED((N_PAD, d_feat), jnp.float32)
        if table_in_spmem
        else None,
        pltpu.VMEM_SHARED((N_PAD, d_feat), jnp.float32),
        [pltpu.SemaphoreType.DMA] * NBUF,
        [pltpu.SemaphoreType.DMA] * NBUF,
    ]

    @functools.partial(
        pl.kernel,
        out_type=jax.ShapeDtypeStruct((NC, N_PAD, d_feat), jnp.float32),
        mesh=_mesh,
        scratch_types=[t for t in scratch if t is not None],
        compiler_params=_sc_params,
    )
    def _prop(g_hbm, src_hbm, dst_hbm, out_hbm, src_v, dst_v, bufs, zero_v, *rest):
        if table_in_spmem:
            g_sh, acc_sh, gsems, ssems = rest
        else:
            acc_sh, gsems, ssems = rest
            g_sh = None
        c = lax.axis_index("core")
        s = lax.axis_index("subcore")
        wid = _worker_id()

        rs = s * ROWS_PER_SUB
        table = g_sh if table_in_spmem else g_hbm
        if table_in_spmem:
            pltpu.sync_copy(
                g_hbm.at[pl.ds(rs, ROWS_PER_SUB)], g_sh.at[pl.ds(rs, ROWS_PER_SUB)]
            )
        _zero_fill(zero_v, d_feat)
        for o, sz in _ZCHUNKS:
            pltpu.sync_copy(zero_v.at[pl.ds(0, sz)], acc_sh.at[pl.ds(rs + o, sz)])
        pltpu.sync_copy(src_hbm.at[wid], src_v)
        pltpu.sync_copy(dst_hbm.at[wid], dst_v)
        plsc.subcore_barrier()

        for b in range(NBUF):
            pltpu.async_copy(table.at[src_v.at[b]], bufs[b], gsems[b])

        @pl.loop(0, NB, step=NBUF)
        def _(j):
            for b in range(NBUF):
                jj = j + b
                pltpu.make_async_copy(table.at[src_v.at[jj]], bufs[b], gsems[b]).wait()
                pltpu.async_copy(bufs[b], acc_sh.at[dst_v.at[jj]], ssems[b], add=True)
            for b in range(NBUF):
                jn = j + b + NBUF

                @pl.when(jn < NB)
                def _():
                    pltpu.make_async_copy(
                        bufs[b], acc_sh.at[dst_v.at[0]], ssems[b]
                    ).wait()
                    pltpu.async_copy(table.at[src_v.at[jn]], bufs[b], gsems[b])

        for b in range(NBUF):
            pltpu.make_async_copy(bufs[b], acc_sh.at[dst_v.at[0]], ssems[b]).wait()

        plsc.subcore_barrier()
        pltpu.sync_copy(
            acc_sh.at[pl.ds(rs, ROWS_PER_SUB)],
            out_hbm.at[c, pl.ds(rs, ROWS_PER_SUB)],
        )

    return _prop


_prop32 = _make_prop(D_OUT, table_in_spmem=True)


def _mm_body(x_ref, w_ref, o_ref):
    o_ref[...] = jnp.dot(
        x_ref[...], w_ref[...], preferred_element_type=jnp.float32,
        precision=lax.Precision.HIGHEST,
    )


def _matmul(x_p, w):
    d_out = w.shape[1]
    return pl.pallas_call(
        _mm_body,
        grid=(GRID,),
        in_specs=[
            pl.BlockSpec((MM_BLK, x_p.shape[1]), lambda i: (i, 0)),
            pl.BlockSpec(w.shape, lambda i: (0, 0)),
        ],
        out_specs=pl.BlockSpec((MM_BLK, d_out), lambda i: (i, 0)),
        out_shape=jax.ShapeDtypeStruct((N_PAD, d_out), jnp.float32),
    )(x_p, w)


def _prep_body(deg_ref, h_ref, dinv_ref, g_ref):
    deg = deg_ref[0, :, 0:1] + deg_ref[1, :, 0:1] + 1.0
    dinv = lax.rsqrt(deg)
    dinv_ref[...] = dinv
    g_ref[...] = h_ref[...] * dinv


def _prep(deg_part, h1):
    return pl.pallas_call(
        _prep_body,
        grid=(GRID,),
        in_specs=[
            pl.BlockSpec((NC, MM_BLK, DEG_W), lambda i: (0, i, 0)),
            pl.BlockSpec((MM_BLK, D_HID), lambda i: (i, 0)),
        ],
        out_specs=[
            pl.BlockSpec((MM_BLK, 1), lambda i: (i, 0)),
            pl.BlockSpec((MM_BLK, D_HID), lambda i: (i, 0)),
        ],
        out_shape=[
            jax.ShapeDtypeStruct((N_PAD, 1), jnp.float32),
            jax.ShapeDtypeStruct((N_PAD, D_HID), jnp.float32),
        ],
    )(deg_part, h1)


def _mid_body(pa_ref, pb_ref, g_ref, dinv_ref, b_ref, w_ref, o_ref):
    p = jnp.concatenate(
        [pa_ref[0] + pa_ref[1], pb_ref[0] + pb_ref[1]], axis=1
    )
    pre = (p + g_ref[...]) * dinv_ref[...] + b_ref[...]
    h = jnp.maximum(pre, 0.0)
    o_ref[...] = jnp.dot(
        h, w_ref[...], preferred_element_type=jnp.float32,
        precision=lax.Precision.HIGHEST,
    ) * dinv_ref[...]


def _mid(part1a, part1b, g1, dinv, b1, w2):
    return pl.pallas_call(
        _mid_body,
        grid=(GRID,),
        in_specs=[
            pl.BlockSpec((NC, MM_BLK, D_OUT), lambda i: (0, i, 0)),
            pl.BlockSpec((NC, MM_BLK, D_OUT), lambda i: (0, i, 0)),
            pl.BlockSpec((MM_BLK, D_HID), lambda i: (i, 0)),
            pl.BlockSpec((MM_BLK, 1), lambda i: (i, 0)),
            pl.BlockSpec((1, D_HID), lambda i: (0, 0)),
            pl.BlockSpec((D_HID, D_OUT), lambda i: (0, 0)),
        ],
        out_specs=pl.BlockSpec((MM_BLK, D_OUT), lambda i: (i, 0)),
        out_shape=jax.ShapeDtypeStruct((N_PAD, D_OUT), jnp.float32),
    )(part1a, part1b, g1, dinv, b1, w2)


def _final_body(p_ref, g_ref, dinv_ref, b_ref, o_ref):
    tot = p_ref[0] + p_ref[1] + g_ref[...]
    o_ref[...] = tot * dinv_ref[...] + b_ref[...]


def _final(part2, g2, dinv, b2):
    return pl.pallas_call(
        _final_body,
        grid=(GRID,),
        in_specs=[
            pl.BlockSpec((NC, MM_BLK, D_OUT), lambda i: (0, i, 0)),
            pl.BlockSpec((MM_BLK, D_OUT), lambda i: (i, 0)),
            pl.BlockSpec((MM_BLK, 1), lambda i: (i, 0)),
            pl.BlockSpec((1, D_OUT), lambda i: (0, 0)),
        ],
        out_specs=pl.BlockSpec((MM_BLK, D_OUT), lambda i: (i, 0)),
        out_shape=jax.ShapeDtypeStruct((N_NODES, D_OUT), jnp.float32),
    )(part2, g2, dinv, b2)


def kernel(x, edge_index, W1, b1, W2, b2):
    ei = edge_index.astype(jnp.int32)
    pad = E_PAD - N_EDGES
    src = jnp.concatenate([ei[0], jnp.zeros((pad,), jnp.int32)]).reshape(NW, NB, BLK)
    dst = jnp.concatenate([ei[1], jnp.full((pad,), N_NODES, jnp.int32)]).reshape(NW, NB, BLK)
    x_p = jnp.concatenate([x, jnp.zeros((N_PAD - N_NODES, D_IN), x.dtype)], axis=0)

    deg_part = _deg_kernel(dst)
    h1 = _matmul(x_p, W1)
    dinv, g1 = _prep(deg_part, h1)
    part1a = _prop32(g1[:, :D_OUT], src, dst)
    part1b = _prop32(g1[:, D_OUT:], src, dst)
    g2 = _mid(part1a, part1b, g1, dinv, b1.reshape(1, D_HID), W2)
    part2 = _prop32(g2, src, dst)
    return _final(part2, g2, dinv, b2.reshape(1, D_OUT))


# speedup vs baseline: 32.4965x; 1.0434x over previous
import functools

import jax
import jax.numpy as jnp
from jax import lax
from jax.experimental import pallas as pl
from jax.experimental.pallas import tpu as pltpu
from jax.experimental.pallas import tpu_sc as plsc

N_NODES = 10000
N_EDGES = 320000
D_IN = 128
D_HID = 64
D_OUT = 32

NC = 2
NS = 16
NW = NC * NS
BLK = 128
NB = 10240 // BLK
E_PAD = NW * NB * BLK
N_PAD = 10016
ROWS_PER_SUB = N_PAD // NS
MM_BLK = 512
GRID = (N_PAD + MM_BLK - 1) // MM_BLK

_ZCHUNKS = [(o, min(BLK, ROWS_PER_SUB - o)) for o in range(0, ROWS_PER_SUB, BLK)]

_mesh = plsc.VectorSubcoreMesh(core_axis_name="core", subcore_axis_name="subcore")
_sc_params = pltpu.CompilerParams(use_tc_tiling_on_sc=False)


def _worker_id():
    return lax.axis_index("core") * NS + lax.axis_index("subcore")


def _zero_fill(ref, width):
    z = jnp.zeros((16,), jnp.float32)

    @pl.loop(0, BLK)
    def _(r):
        for t in range(width // 16):
            ref[r, pl.ds(t * 16, 16)] = z


DEG_W = 16


@functools.partial(
    pl.kernel,
    out_type=jax.ShapeDtypeStruct((NC, N_PAD, DEG_W), jnp.float32),
    mesh=_mesh,
    scratch_types=[
        pltpu.VMEM((NB, BLK), jnp.int32),
        pltpu.VMEM((BLK, DEG_W), jnp.float32),
        pltpu.VMEM((BLK, DEG_W), jnp.float32),
        pltpu.VMEM_SHARED((N_PAD, DEG_W), jnp.float32),
    ],
    compiler_params=_sc_params,
)
def _deg_kernel(dst_hbm, out_hbm, idx_v, ones_v, zero_v, acc_sh):
    c = lax.axis_index("core")
    s = lax.axis_index("subcore")
    wid = _worker_id()

    one = jnp.ones((16,), jnp.float32)

    @pl.loop(0, BLK)
    def _(r):
        ones_v[r, pl.ds(0, 16)] = one

    _zero_fill(zero_v, DEG_W)
    for o, sz in _ZCHUNKS:
        pltpu.sync_copy(
            zero_v.at[pl.ds(0, sz)], acc_sh.at[pl.ds(s * ROWS_PER_SUB + o, sz)]
        )
    pltpu.sync_copy(dst_hbm.at[wid], idx_v)
    plsc.subcore_barrier()

    @pl.loop(0, NB)
    def _(j):
        pltpu.sync_copy(ones_v, acc_sh.at[idx_v.at[j]], add=True)

    plsc.subcore_barrier()
    pltpu.sync_copy(
        acc_sh.at[pl.ds(s * ROWS_PER_SUB, ROWS_PER_SUB)],
        out_hbm.at[c, pl.ds(s * ROWS_PER_SUB, ROWS_PER_SUB)],
    )


NBUF = 8


def _make_prop(d_feat, table_in_spmem):
    scratch = [
        pltpu.VMEM((NB, BLK), jnp.int32),
        pltpu.VMEM((NB, BLK), jnp.int32),
        [pltpu.VMEM((BLK, d_feat), jnp.float32)] * NBUF,
        pltpu.VMEM((BLK, d_feat), jnp.float32),
        pltpu.VMEM_SHARED((N_PAD, d_feat), jnp.float32)
        if table_in_spmem
        else None,
        pltpu.VMEM_SHARED((N_PAD, d_feat), jnp.float32),
        [pltpu.SemaphoreType.DMA] * NBUF,
        [pltpu.SemaphoreType.DMA] * NBUF,
    ]

    @functools.partial(
        pl.kernel,
        out_type=jax.ShapeDtypeStruct((NC, N_PAD, d_feat), jnp.float32),
        mesh=_mesh,
        scratch_types=[t for t in scratch if t is not None],
        compiler_params=_sc_params,
    )
    def _prop(g_hbm, src_hbm, dst_hbm, out_hbm, src_v, dst_v, bufs, zero_v, *rest):
        if table_in_spmem:
            g_sh, acc_sh, gsems, ssems = rest
        else:
            acc_sh, gsems, ssems = rest
            g_sh = None
        c = lax.axis_index("core")
        s = lax.axis_index("subcore")
        wid = _worker_id()

        rs = s * ROWS_PER_SUB
        table = g_sh if table_in_spmem else g_hbm
        if table_in_spmem:
            pltpu.sync_copy(
                g_hbm.at[pl.ds(rs, ROWS_PER_SUB)], g_sh.at[pl.ds(rs, ROWS_PER_SUB)]
            )
        _zero_fill(zero_v, d_feat)
        for o, sz in _ZCHUNKS:
            pltpu.sync_copy(zero_v.at[pl.ds(0, sz)], acc_sh.at[pl.ds(rs + o, sz)])
        pltpu.sync_copy(src_hbm.at[wid], src_v)
        pltpu.sync_copy(dst_hbm.at[wid], dst_v)
        plsc.subcore_barrier()

        for b in range(NBUF):
            pltpu.async_copy(table.at[src_v.at[b]], bufs[b], gsems[b])

        @pl.loop(0, NB, step=NBUF)
        def _(j):
            for b in range(NBUF):
                jj = j + b
                pltpu.make_async_copy(table.at[src_v.at[jj]], bufs[b], gsems[b]).wait()
                pltpu.async_copy(bufs[b], acc_sh.at[dst_v.at[jj]], ssems[b], add=True)
            for b in range(NBUF):
                jn = j + b + NBUF

                @pl.when(jn < NB)
                def _():
                    pltpu.make_async_copy(
                        bufs[b], acc_sh.at[dst_v.at[0]], ssems[b]
                    ).wait()
                    pltpu.async_copy(table.at[src_v.at[jn]], bufs[b], gsems[b])

        for b in range(NBUF):
            pltpu.make_async_copy(bufs[b], acc_sh.at[dst_v.at[0]], ssems[b]).wait()

        plsc.subcore_barrier()
        pltpu.sync_copy(
            acc_sh.at[pl.ds(rs, ROWS_PER_SUB)],
            out_hbm.at[c, pl.ds(rs, ROWS_PER_SUB)],
        )

    return _prop


_prop32 = _make_prop(D_OUT, table_in_spmem=True)


def _mm_body(x_ref, w_ref, o_ref):
    o_ref[...] = jnp.dot(
        x_ref[...], w_ref[...], preferred_element_type=jnp.float32,
        precision=lax.Precision.HIGHEST,
    )


def _matmul(x_p, w):
    d_out = w.shape[1]
    return pl.pallas_call(
        _mm_body,
        grid=(GRID,),
        in_specs=[
            pl.BlockSpec((MM_BLK, x_p.shape[1]), lambda i: (i, 0)),
            pl.BlockSpec(w.shape, lambda i: (0, 0)),
        ],
        out_specs=pl.BlockSpec((MM_BLK, d_out), lambda i: (i, 0)),
        out_shape=jax.ShapeDtypeStruct((N_PAD, d_out), jnp.float32),
    )(x_p, w)


def _prep_body(deg_ref, h_ref, dinv_ref, g_ref):
    deg = deg_ref[0, :, 0:1] + deg_ref[1, :, 0:1] + 1.0
    dinv = lax.rsqrt(deg)
    dinv_ref[...] = dinv
    g_ref[...] = h_ref[...] * dinv


def _prep(deg_part, h1):
    return pl.pallas_call(
        _prep_body,
        grid=(GRID,),
        in_specs=[
            pl.BlockSpec((NC, MM_BLK, DEG_W), lambda i: (0, i, 0)),
            pl.BlockSpec((MM_BLK, D_HID), lambda i: (i, 0)),
        ],
        out_specs=[
            pl.BlockSpec((MM_BLK, 1), lambda i: (i, 0)),
            pl.BlockSpec((MM_BLK, D_HID), lambda i: (i, 0)),
        ],
        out_shape=[
            jax.ShapeDtypeStruct((N_PAD, 1), jnp.float32),
            jax.ShapeDtypeStruct((N_PAD, D_HID), jnp.float32),
        ],
    )(deg_part, h1)


def _mid_body(pa_ref, pb_ref, g_ref, dinv_ref, b_ref, w_ref, o_ref):
    p = jnp.concatenate(
        [pa_ref[0] + pa_ref[1], pb_ref[0] + pb_ref[1]], axis=1
    )
    pre = (p + g_ref[...]) * dinv_ref[...] + b_ref[...]
    h = jnp.maximum(pre, 0.0)
    o_ref[...] = jnp.dot(
        h, w_ref[...], preferred_element_type=jnp.float32,
        precision=lax.Precision.HIGHEST,
    ) * dinv_ref[...]


def _mid(part1a, part1b, g1, dinv, b1, w2):
    return pl.pallas_call(
        _mid_body,
        grid=(GRID,),
        in_specs=[
            pl.BlockSpec((NC, MM_BLK, D_OUT), lambda i: (0, i, 0)),
            pl.BlockSpec((NC, MM_BLK, D_OUT), lambda i: (0, i, 0)),
            pl.BlockSpec((MM_BLK, D_HID), lambda i: (i, 0)),
            pl.BlockSpec((MM_BLK, 1), lambda i: (i, 0)),
            pl.BlockSpec((1, D_HID), lambda i: (0, 0)),
            pl.BlockSpec((D_HID, D_OUT), lambda i: (0, 0)),
        ],
        out_specs=pl.BlockSpec((MM_BLK, D_OUT), lambda i: (i, 0)),
        out_shape=jax.ShapeDtypeStruct((N_PAD, D_OUT), jnp.float32),
    )(part1a, part1b, g1, dinv, b1, w2)


def _final_body(p_ref, g_ref, dinv_ref, b_ref, o_ref):
    tot = p_ref[0] + p_ref[1] + g_ref[...]
    o_ref[...] = tot * dinv_ref[...] + b_ref[...]


def _final(part2, g2, dinv, b2):
    return pl.pallas_call(
        _final_body,
        grid=(GRID,),
        in_specs=[
            pl.BlockSpec((NC, MM_BLK, D_OUT), lambda i: (0, i, 0)),
            pl.BlockSpec((MM_BLK, D_OUT), lambda i: (i, 0)),
            pl.BlockSpec((MM_BLK, 1), lambda i: (i, 0)),
            pl.BlockSpec((1, D_OUT), lambda i: (0, 0)),
        ],
        out_specs=pl.BlockSpec((MM_BLK, D_OUT), lambda i: (i, 0)),
        out_shape=jax.ShapeDtypeStruct((N_NODES, D_OUT), jnp.float32),
    )(part2, g2, dinv, b2)


def kernel(x, edge_index, W1, b1, W2, b2):
    ei = edge_index.astype(jnp.int32)
    pad = E_PAD - N_EDGES
    src = jnp.concatenate([ei[0], jnp.zeros((pad,), jnp.int32)]).reshape(NW, NB, BLK)
    dst = jnp.concatenate([ei[1], jnp.full((pad,), N_NODES, jnp.int32)]).reshape(NW, NB, BLK)
    x_p = jnp.concatenate([x, jnp.zeros((N_PAD - N_NODES, D_IN), x.dtype)], axis=0)

    deg_part = _deg_kernel(dst)
    h1 = _matmul(x_p, W1)
    dinv, g1 = _prep(deg_part, h1)
    part1a = _prop32(g1[:, :D_OUT], src, dst)
    part1b = _prop32(g1[:, D_OUT:], src, dst)
    g2 = _mid(part1a, part1b, g1, dinv, b1.reshape(1, D_HID), W2)
    part2 = _prop32(g2, src, dst)
    return _final(part2, g2, dinv, b2.reshape(1, D_OUT))


# speedup vs baseline: 33.1255x; 1.0194x over previous
import functools

import jax
import jax.numpy as jnp
from jax import lax
from jax.experimental import pallas as pl
from jax.experimental.pallas import tpu as pltpu
from jax.experimental.pallas import tpu_sc as plsc

N_NODES = 10000
N_EDGES = 320000
D_IN = 128
D_HID = 64
D_OUT = 32

NC = 2
NS = 16
NW = NC * NS
BLK = 128
NB = 10240 // BLK
E_PAD = NW * NB * BLK
N_PAD = 10016
ROWS_PER_SUB = N_PAD // NS
MM_BLK = 512
GRID = (N_PAD + MM_BLK - 1) // MM_BLK

_ZCHUNKS = [(o, min(BLK, ROWS_PER_SUB - o)) for o in range(0, ROWS_PER_SUB, BLK)]

_mesh = plsc.VectorSubcoreMesh(core_axis_name="core", subcore_axis_name="subcore")
_sc_params = pltpu.CompilerParams(use_tc_tiling_on_sc=False)


def _worker_id():
    return lax.axis_index("core") * NS + lax.axis_index("subcore")


def _zero_fill(ref, width):
    z = jnp.zeros((16,), jnp.float32)

    @pl.loop(0, BLK)
    def _(r):
        for t in range(width // 16):
            ref[r, pl.ds(t * 16, 16)] = z


DEG_W = 16


@functools.partial(
    pl.kernel,
    out_type=jax.ShapeDtypeStruct((NC, N_PAD, DEG_W), jnp.float32),
    mesh=_mesh,
    scratch_types=[
        pltpu.VMEM((NB, BLK), jnp.int32),
        pltpu.VMEM((BLK, DEG_W), jnp.float32),
        pltpu.VMEM((BLK, DEG_W), jnp.float32),
        pltpu.VMEM_SHARED((N_PAD, DEG_W), jnp.float32),
    ],
    compiler_params=_sc_params,
)
def _deg_kernel(dst_hbm, out_hbm, idx_v, ones_v, zero_v, acc_sh):
    c = lax.axis_index("core")
    s = lax.axis_index("subcore")
    wid = _worker_id()

    one = jnp.ones((16,), jnp.float32)

    @pl.loop(0, BLK)
    def _(r):
        ones_v[r, pl.ds(0, 16)] = one

    _zero_fill(zero_v, DEG_W)
    for o, sz in _ZCHUNKS:
        pltpu.sync_copy(
            zero_v.at[pl.ds(0, sz)], acc_sh.at[pl.ds(s * ROWS_PER_SUB + o, sz)]
        )
    pltpu.sync_copy(dst_hbm.at[wid], idx_v)
    plsc.subcore_barrier()

    @pl.loop(0, NB)
    def _(j):
        pltpu.sync_copy(ones_v, acc_sh.at[idx_v.at[j]], add=True)

    plsc.subcore_barrier()
    pltpu.sync_copy(
        acc_sh.at[pl.ds(s * ROWS_PER_SUB, ROWS_PER_SUB)],
        out_hbm.at[c, pl.ds(s * ROWS_PER_SUB, ROWS_PER_SUB)],
    )


NBUF = 8


def _make_prop(d_feat, table_in_spmem):
    scratch = [
        pltpu.VMEM((NB, BLK), jnp.int32),
        pltpu.VMEM((NB, BLK), jnp.int32),
        [pltpu.VMEM((BLK, d_feat), jnp.float32)] * NBUF,
        pltpu.VMEM((BLK, d_feat), jnp.float32),
        pltpu.VMEM_SHARED((N_PAD, d_feat), jnp.float32)
        if table_in_spmem
        else None,
        pltpu.VMEM_SHARED((N_PAD, d_feat), jnp.float32),
        [pltpu.SemaphoreType.DMA] * NBUF,
        [pltpu.SemaphoreType.DMA] * NBUF,
    ]

    @functools.partial(
        pl.kernel,
        out_type=jax.ShapeDtypeStruct((NC, N_PAD, d_feat), jnp.float32),
        mesh=_mesh,
        scratch_types=[t for t in scratch if t is not None],
        compiler_params=_sc_params,
    )
    def _prop(g_hbm, src_hbm, dst_hbm, out_hbm, src_v, dst_v, bufs, zero_v, *rest):
        if table_in_spmem:
            g_sh, acc_sh, gsems, ssems = rest
        else:
            acc_sh, gsems, ssems = rest
            g_sh = None
        c = lax.axis_index("core")
        s = lax.axis_index("subcore")
        wid = _worker_id()

        rs = s * ROWS_PER_SUB
        table = g_sh if table_in_spmem else g_hbm
        if table_in_spmem:
            pltpu.sync_copy(
                g_hbm.at[pl.ds(rs, ROWS_PER_SUB)], g_sh.at[pl.ds(rs, ROWS_PER_SUB)]
            )
        _zero_fill(zero_v, d_feat)
        for o, sz in _ZCHUNKS:
            pltpu.sync_copy(zero_v.at[pl.ds(0, sz)], acc_sh.at[pl.ds(rs + o, sz)])
        pltpu.sync_copy(src_hbm.at[wid], src_v)
        pltpu.sync_copy(dst_hbm.at[wid], dst_v)
        plsc.subcore_barrier()

        for b in range(NBUF):
            pltpu.async_copy(table.at[src_v.at[b]], bufs[b], gsems[b])

        @pl.loop(0, NB, step=NBUF)
        def _(j):
            for b in range(NBUF):
                jj = j + b
                pltpu.make_async_copy(table.at[src_v.at[jj]], bufs[b], gsems[b]).wait()
                pltpu.async_copy(bufs[b], acc_sh.at[dst_v.at[jj]], ssems[b], add=True)
            for b in range(NBUF):
                jn = j + b + NBUF

                @pl.when(jn < NB)
                def _():
                    pltpu.make_async_copy(
                        bufs[b], acc_sh.at[dst_v.at[0]], ssems[b]
                    ).wait()
                    pltpu.async_copy(table.at[src_v.at[jn]], bufs[b], gsems[b])

        for b in range(NBUF):
            pltpu.make_async_copy(bufs[b], acc_sh.at[dst_v.at[0]], ssems[b]).wait()

        plsc.subcore_barrier()
        pltpu.sync_copy(
            acc_sh.at[pl.ds(rs, ROWS_PER_SUB)],
            out_hbm.at[c, pl.ds(rs, ROWS_PER_SUB)],
        )

    return _prop


_prop32 = _make_prop(D_OUT, table_in_spmem=True)


def _mm_body(x_ref, w_ref, o_ref):
    o_ref[...] = jnp.dot(
        x_ref[...], w_ref[...], preferred_element_type=jnp.float32,
        precision=lax.Precision.HIGHEST,
    )


def _matmul(x_p, w):
    d_out = w.shape[1]
    return pl.pallas_call(
        _mm_body,
        grid=(GRID,),
        in_specs=[
            pl.BlockSpec((MM_BLK, x_p.shape[1]), lambda i: (i, 0)),
            pl.BlockSpec(w.shape, lambda i: (0, 0)),
        ],
        out_specs=pl.BlockSpec((MM_BLK, d_out), lambda i: (i, 0)),
        out_shape=jax.ShapeDtypeStruct((N_PAD, d_out), jnp.float32),
    )(x_p, w)


def _prep_body(deg_ref, x_ref, w_ref, dinv_ref, ga_ref, gb_ref):
    h = jnp.dot(
        x_ref[...], w_ref[...], preferred_element_type=jnp.float32,
        precision=lax.Precision.HIGHEST,
    )
    deg = deg_ref[0, :, 0:1] + deg_ref[1, :, 0:1] + 1.0
    dinv = lax.rsqrt(deg)
    dinv_ref[...] = dinv
    g = h * dinv
    ga_ref[...] = g[:, :D_OUT]
    gb_ref[...] = g[:, D_OUT:]


def _prep(deg_part, x_p, w1):
    return pl.pallas_call(
        _prep_body,
        grid=(GRID,),
        in_specs=[
            pl.BlockSpec((NC, MM_BLK, DEG_W), lambda i: (0, i, 0)),
            pl.BlockSpec((MM_BLK, D_IN), lambda i: (i, 0)),
            pl.BlockSpec((D_IN, D_HID), lambda i: (0, 0)),
        ],
        out_specs=[
            pl.BlockSpec((MM_BLK, 1), lambda i: (i, 0)),
            pl.BlockSpec((MM_BLK, D_OUT), lambda i: (i, 0)),
            pl.BlockSpec((MM_BLK, D_OUT), lambda i: (i, 0)),
        ],
        out_shape=[
            jax.ShapeDtypeStruct((N_PAD, 1), jnp.float32),
            jax.ShapeDtypeStruct((N_PAD, D_OUT), jnp.float32),
            jax.ShapeDtypeStruct((N_PAD, D_OUT), jnp.float32),
        ],
    )(deg_part, x_p, w1)


def _mid_body(pa_ref, pb_ref, ga_ref, gb_ref, dinv_ref, b_ref, w_ref, o_ref):
    p = jnp.concatenate(
        [pa_ref[0] + pa_ref[1] + ga_ref[...], pb_ref[0] + pb_ref[1] + gb_ref[...]],
        axis=1,
    )
    pre = p * dinv_ref[...] + b_ref[...]
    h = jnp.maximum(pre, 0.0)
    o_ref[...] = jnp.dot(
        h, w_ref[...], preferred_element_type=jnp.float32,
        precision=lax.Precision.HIGHEST,
    ) * dinv_ref[...]


def _mid(part1a, part1b, g1a, g1b, dinv, b1, w2):
    return pl.pallas_call(
        _mid_body,
        grid=(GRID,),
        in_specs=[
            pl.BlockSpec((NC, MM_BLK, D_OUT), lambda i: (0, i, 0)),
            pl.BlockSpec((NC, MM_BLK, D_OUT), lambda i: (0, i, 0)),
            pl.BlockSpec((MM_BLK, D_OUT), lambda i: (i, 0)),
            pl.BlockSpec((MM_BLK, D_OUT), lambda i: (i, 0)),
            pl.BlockSpec((MM_BLK, 1), lambda i: (i, 0)),
            pl.BlockSpec((1, D_HID), lambda i: (0, 0)),
            pl.BlockSpec((D_HID, D_OUT), lambda i: (0, 0)),
        ],
        out_specs=pl.BlockSpec((MM_BLK, D_OUT), lambda i: (i, 0)),
        out_shape=jax.ShapeDtypeStruct((N_PAD, D_OUT), jnp.float32),
    )(part1a, part1b, g1a, g1b, dinv, b1, w2)


def _final_body(p_ref, g_ref, dinv_ref, b_ref, o_ref):
    tot = p_ref[0] + p_ref[1] + g_ref[...]
    o_ref[...] = tot * dinv_ref[...] + b_ref[...]


def _final(part2, g2, dinv, b2):
    return pl.pallas_call(
        _final_body,
        grid=(GRID,),
        in_specs=[
            pl.BlockSpec((NC, MM_BLK, D_OUT), lambda i: (0, i, 0)),
            pl.BlockSpec((MM_BLK, D_OUT), lambda i: (i, 0)),
            pl.BlockSpec((MM_BLK, 1), lambda i: (i, 0)),
            pl.BlockSpec((1, D_OUT), lambda i: (0, 0)),
        ],
        out_specs=pl.BlockSpec((MM_BLK, D_OUT), lambda i: (i, 0)),
        out_shape=jax.ShapeDtypeStruct((N_NODES, D_OUT), jnp.float32),
    )(part2, g2, dinv, b2)


def kernel(x, edge_index, W1, b1, W2, b2):
    ei = edge_index.astype(jnp.int32)
    pad = E_PAD - N_EDGES
    src = jnp.concatenate([ei[0], jnp.zeros((pad,), jnp.int32)]).reshape(NW, NB, BLK)
    dst = jnp.concatenate([ei[1], jnp.full((pad,), N_NODES, jnp.int32)]).reshape(NW, NB, BLK)
    x_p = jnp.concatenate([x, jnp.zeros((N_PAD - N_NODES, D_IN), x.dtype)], axis=0)

    deg_part = _deg_kernel(dst)
    dinv, g1a, g1b = _prep(deg_part, x_p, W1)
    part1a = _prop32(g1a, src, dst)
    part1b = _prop32(g1b, src, dst)
    g2 = _mid(part1a, part1b, g1a, g1b, dinv, b1.reshape(1, D_HID), W2)
    part2 = _prop32(g2, src, dst)
    return _final(part2, g2, dinv, b2.reshape(1, D_OUT))


# speedup vs baseline: 33.1335x; 1.0002x over previous
import functools

import jax
import jax.numpy as jnp
from jax import lax
from jax.experimental import pallas as pl
from jax.experimental.pallas import tpu as pltpu
from jax.experimental.pallas import tpu_sc as plsc

N_NODES = 10000
N_EDGES = 320000
D_IN = 128
D_HID = 64
D_OUT = 32

NC = 2
NS = 16
NW = NC * NS
BLK = 128
NB = 10240 // BLK
E_PAD = NW * NB * BLK
N_PAD = 10016
ROWS_PER_SUB = N_PAD // NS
MM_BLK = 512
GRID = (N_PAD + MM_BLK - 1) // MM_BLK

_ZCHUNKS = [(o, min(BLK, ROWS_PER_SUB - o)) for o in range(0, ROWS_PER_SUB, BLK)]

_mesh = plsc.VectorSubcoreMesh(core_axis_name="core", subcore_axis_name="subcore")
_sc_params = pltpu.CompilerParams(use_tc_tiling_on_sc=False)


def _worker_id():
    return lax.axis_index("core") * NS + lax.axis_index("subcore")


def _zero_fill(ref, width):
    z = jnp.zeros((16,), jnp.float32)

    @pl.loop(0, BLK)
    def _(r):
        for t in range(width // 16):
            ref[r, pl.ds(t * 16, 16)] = z


DEG_W = 16


@functools.partial(
    pl.kernel,
    out_type=jax.ShapeDtypeStruct((NC, N_PAD, DEG_W), jnp.float32),
    mesh=_mesh,
    scratch_types=[
        pltpu.VMEM((NB, BLK), jnp.int32),
        pltpu.VMEM((BLK, DEG_W), jnp.float32),
        pltpu.VMEM((BLK, DEG_W), jnp.float32),
        pltpu.VMEM_SHARED((N_PAD, DEG_W), jnp.float32),
    ],
    compiler_params=_sc_params,
)
def _deg_kernel(dst_hbm, out_hbm, idx_v, ones_v, zero_v, acc_sh):
    c = lax.axis_index("core")
    s = lax.axis_index("subcore")
    wid = _worker_id()

    one = jnp.ones((16,), jnp.float32)

    @pl.loop(0, BLK)
    def _(r):
        ones_v[r, pl.ds(0, 16)] = one

    _zero_fill(zero_v, DEG_W)
    for o, sz in _ZCHUNKS:
        pltpu.sync_copy(
            zero_v.at[pl.ds(0, sz)], acc_sh.at[pl.ds(s * ROWS_PER_SUB + o, sz)]
        )
    pltpu.sync_copy(dst_hbm.at[wid], idx_v)
    plsc.subcore_barrier()

    @pl.loop(0, NB)
    def _(j):
        pltpu.sync_copy(ones_v, acc_sh.at[idx_v.at[j]], add=True)

    plsc.subcore_barrier()
    pltpu.sync_copy(
        acc_sh.at[pl.ds(s * ROWS_PER_SUB, ROWS_PER_SUB)],
        out_hbm.at[c, pl.ds(s * ROWS_PER_SUB, ROWS_PER_SUB)],
    )


NBUF = 8


def _make_prop(d_feat, table_in_spmem):
    scratch = [
        pltpu.VMEM((NB, BLK), jnp.int32),
        pltpu.VMEM((NB, BLK), jnp.int32),
        [pltpu.VMEM((BLK, d_feat), jnp.float32)] * NBUF,
        pltpu.VMEM((BLK, d_feat), jnp.float32),
        pltpu.VMEM_SHARED((N_PAD, d_feat), jnp.float32)
        if table_in_spmem
        else None,
        pltpu.VMEM_SHARED((N_PAD, d_feat), jnp.float32),
        [pltpu.SemaphoreType.DMA] * NBUF,
        [pltpu.SemaphoreType.DMA] * NBUF,
    ]

    @functools.partial(
        pl.kernel,
        out_type=jax.ShapeDtypeStruct((NC, N_PAD, d_feat), jnp.float32),
        mesh=_mesh,
        scratch_types=[t for t in scratch if t is not None],
        compiler_params=_sc_params,
    )
    def _prop(g_hbm, src_hbm, dst_hbm, out_hbm, src_v, dst_v, bufs, zero_v, *rest):
        if table_in_spmem:
            g_sh, acc_sh, gsems, ssems = rest
        else:
            acc_sh, gsems, ssems = rest
            g_sh = None
        c = lax.axis_index("core")
        s = lax.axis_index("subcore")
        wid = _worker_id()

        rs = s * ROWS_PER_SUB
        table = g_sh if table_in_spmem else g_hbm
        if table_in_spmem:
            pltpu.sync_copy(
                g_hbm.at[pl.ds(rs, ROWS_PER_SUB)], g_sh.at[pl.ds(rs, ROWS_PER_SUB)]
            )
        _zero_fill(zero_v, d_feat)
        for o, sz in _ZCHUNKS:
            pltpu.sync_copy(zero_v.at[pl.ds(0, sz)], acc_sh.at[pl.ds(rs + o, sz)])
        pltpu.sync_copy(src_hbm.at[wid], src_v)
        pltpu.sync_copy(dst_hbm.at[wid], dst_v)
        plsc.subcore_barrier()

        for b in range(NBUF):
            pltpu.async_copy(table.at[src_v.at[b]], bufs[b], gsems[b])

        @pl.loop(0, NB, step=NBUF)
        def _(j):
            for b in range(NBUF):
                jj = j + b
                pltpu.make_async_copy(table.at[src_v.at[jj]], bufs[b], gsems[b]).wait()
                pltpu.async_copy(bufs[b], acc_sh.at[dst_v.at[jj]], ssems[b], add=True)
            for b in range(NBUF):
                jn = j + b + NBUF

                @pl.when(jn < NB)
                def _():
                    pltpu.make_async_copy(
                        bufs[b], acc_sh.at[dst_v.at[0]], ssems[b]
                    ).wait()
                    pltpu.async_copy(table.at[src_v.at[jn]], bufs[b], gsems[b])

        for b in range(NBUF):
            pltpu.make_async_copy(bufs[b], acc_sh.at[dst_v.at[0]], ssems[b]).wait()

        plsc.subcore_barrier()
        pltpu.sync_copy(
            acc_sh.at[pl.ds(rs, ROWS_PER_SUB)],
            out_hbm.at[c, pl.ds(rs, ROWS_PER_SUB)],
        )

    return _prop


_prop32 = _make_prop(D_OUT, table_in_spmem=True)


def _prep_body(deg_ref, x_ref, w_ref, dinv_ref, ga_ref, gb_ref):
    h = jnp.dot(
        x_ref[...], w_ref[...], preferred_element_type=jnp.float32,
        precision=lax.Precision.HIGHEST,
    )
    deg = deg_ref[0, :, 0:1] + deg_ref[1, :, 0:1] + 1.0
    dinv = lax.rsqrt(deg)
    dinv_ref[...] = dinv
    g = h * dinv
    ga_ref[...] = g[:, :D_OUT]
    gb_ref[...] = g[:, D_OUT:]


def _prep(deg_part, x_p, w1):
    return pl.pallas_call(
        _prep_body,
        grid=(GRID,),
        in_specs=[
            pl.BlockSpec((NC, MM_BLK, DEG_W), lambda i: (0, i, 0)),
            pl.BlockSpec((MM_BLK, D_IN), lambda i: (i, 0)),
            pl.BlockSpec((D_IN, D_HID), lambda i: (0, 0)),
        ],
        out_specs=[
            pl.BlockSpec((MM_BLK, 1), lambda i: (i, 0)),
            pl.BlockSpec((MM_BLK, D_OUT), lambda i: (i, 0)),
            pl.BlockSpec((MM_BLK, D_OUT), lambda i: (i, 0)),
        ],
        out_shape=[
            jax.ShapeDtypeStruct((N_PAD, 1), jnp.float32),
            jax.ShapeDtypeStruct((N_PAD, D_OUT), jnp.float32),
            jax.ShapeDtypeStruct((N_PAD, D_OUT), jnp.float32),
        ],
    )(deg_part, x_p, w1)


def _mid_body(pa_ref, pb_ref, ga_ref, gb_ref, dinv_ref, b_ref, w_ref, o_ref):
    p = jnp.concatenate(
        [pa_ref[0] + pa_ref[1] + ga_ref[...], pb_ref[0] + pb_ref[1] + gb_ref[...]],
        axis=1,
    )
    pre = p * dinv_ref[...] + b_ref[...]
    h = jnp.maximum(pre, 0.0)
    o_ref[...] = jnp.dot(
        h, w_ref[...], preferred_element_type=jnp.float32,
        precision=lax.Precision.HIGHEST,
    ) * dinv_ref[...]


def _mid(part1a, part1b, g1a, g1b, dinv, b1, w2):
    return pl.pallas_call(
        _mid_body,
        grid=(GRID,),
        in_specs=[
            pl.BlockSpec((NC, MM_BLK, D_OUT), lambda i: (0, i, 0)),
            pl.BlockSpec((NC, MM_BLK, D_OUT), lambda i: (0, i, 0)),
            pl.BlockSpec((MM_BLK, D_OUT), lambda i: (i, 0)),
            pl.BlockSpec((MM_BLK, D_OUT), lambda i: (i, 0)),
            pl.BlockSpec((MM_BLK, 1), lambda i: (i, 0)),
            pl.BlockSpec((1, D_HID), lambda i: (0, 0)),
            pl.BlockSpec((D_HID, D_OUT), lambda i: (0, 0)),
        ],
        out_specs=pl.BlockSpec((MM_BLK, D_OUT), lambda i: (i, 0)),
        out_shape=jax.ShapeDtypeStruct((N_PAD, D_OUT), jnp.float32),
    )(part1a, part1b, g1a, g1b, dinv, b1, w2)


def _final_body(p_ref, g_ref, dinv_ref, b_ref, o_ref):
    tot = p_ref[0] + p_ref[1] + g_ref[...]
    o_ref[...] = tot * dinv_ref[...] + b_ref[...]


def _final(part2, g2, dinv, b2):
    return pl.pallas_call(
        _final_body,
        grid=(GRID,),
        in_specs=[
            pl.BlockSpec((NC, MM_BLK, D_OUT), lambda i: (0, i, 0)),
            pl.BlockSpec((MM_BLK, D_OUT), lambda i: (i, 0)),
            pl.BlockSpec((MM_BLK, 1), lambda i: (i, 0)),
            pl.BlockSpec((1, D_OUT), lambda i: (0, 0)),
        ],
        out_specs=pl.BlockSpec((MM_BLK, D_OUT), lambda i: (i, 0)),
        out_shape=jax.ShapeDtypeStruct((N_NODES, D_OUT), jnp.float32),
    )(part2, g2, dinv, b2)


def kernel(x, edge_index, W1, b1, W2, b2):
    ei = edge_index.astype(jnp.int32)
    pad = E_PAD - N_EDGES
    src = jnp.concatenate([ei[0], jnp.zeros((pad,), jnp.int32)]).reshape(NW, NB, BLK)
    dst = jnp.concatenate([ei[1], jnp.full((pad,), N_NODES, jnp.int32)]).reshape(NW, NB, BLK)
    x_p = jnp.concatenate([x, jnp.zeros((N_PAD - N_NODES, D_IN), x.dtype)], axis=0)

    deg_part = _deg_kernel(dst)
    dinv, g1a, g1b = _prep(deg_part, x_p, W1)
    part1a = _prop32(g1a, src, dst)
    part1b = _prop32(g1b, src, dst)
    g2 = _mid(part1a, part1b, g1a, g1b, dinv, b1.reshape(1, D_HID), W2)
    part2 = _prop32(g2, src, dst)
    return _final(part2, g2, dinv, b2.reshape(1, D_OUT))
